# baseline (speedup 1.0000x reference)
import functools

import jax
import jax.numpy as jnp
from jax import lax
from jax.experimental import pallas as pl
from jax.experimental.pallas import tpu as pltpu
from jax.experimental.pallas import tpu_sc as plsc

N = 10000
D = 128
NP = 10240
BN = 2048
NW = 32
C = 128
CH0_FRAC_NUM = 12
CH0_FRAC_DEN = 16
ROWS = NP // 16

_MESH = plsc.VectorSubcoreMesh(core_axis_name="c", subcore_axis_name="s")


def _sc_degree(dst_p, ones_mat, zeros_mat, ch):
    @functools.partial(
        pl.kernel,
        out_type=jax.ShapeDtypeStruct((2, NP, D), jnp.float32),
        mesh=_MESH,
        scratch_types=[
            pltpu.VMEM((1, C), jnp.int32),
            pltpu.VMEM((1, C), jnp.int32),
            pltpu.VMEM((C, D), jnp.float32),
            pltpu.VMEM_SHARED((NP, D), jnp.float32),
            pltpu.SemaphoreType.DMA,
            pltpu.SemaphoreType.DMA,
        ],
    )
    def k(dst_hbm, ones_hbm, zeros_hbm, out_hbm,
          idx_a, idx_b, ones_v, acc, sem_a, sem_b):
        c = lax.axis_index("c")
        s = lax.axis_index("s")
        wid = s * 2 + c
        pltpu.sync_copy(ones_hbm, ones_v)
        pltpu.sync_copy(zeros_hbm.at[pl.ds(s * ROWS, ROWS)],
                        acc.at[pl.ds(s * ROWS, ROWS)])
        plsc.subcore_barrier()

        pltpu.sync_copy(dst_hbm.at[wid, 0], idx_a)
        pltpu.async_copy(ones_v, acc.at[idx_a.at[0]], sem_a, add=True)
        pltpu.sync_copy(dst_hbm.at[wid, 1], idx_b)
        pltpu.async_copy(ones_v, acc.at[idx_b.at[0]], sem_b, add=True)

        @pl.loop(0, ch - 2, step=2)
        def _(j):
            pltpu.make_async_copy(ones_v, acc.at[idx_a.at[0]], sem_a).wait()
            pltpu.sync_copy(dst_hbm.at[wid, j + 2], idx_a)
            pltpu.async_copy(ones_v, acc.at[idx_a.at[0]], sem_a, add=True)
            pltpu.make_async_copy(ones_v, acc.at[idx_b.at[0]], sem_b).wait()
            pltpu.sync_copy(dst_hbm.at[wid, j + 3], idx_b)
            pltpu.async_copy(ones_v, acc.at[idx_b.at[0]], sem_b, add=True)

        pltpu.make_async_copy(ones_v, acc.at[idx_a.at[0]], sem_a).wait()
        pltpu.make_async_copy(ones_v, acc.at[idx_b.at[0]], sem_b).wait()
        plsc.subcore_barrier()
        pltpu.sync_copy(acc.at[pl.ds(s * ROWS, ROWS)],
                        out_hbm.at[c, pl.ds(s * ROWS, ROWS)])

    return k(dst_p, ones_mat, zeros_mat)


def _sc_aggregate(h, e0_p, e1_p, zeros_mat, ch0, ch1):
    @functools.partial(
        pl.kernel,
        out_type=jax.ShapeDtypeStruct((2, NP, D), jnp.float32),
        mesh=_MESH,
        scratch_types=[
            pltpu.VMEM((2, C), jnp.int32),
            pltpu.VMEM((2, C), jnp.int32),
            pltpu.VMEM((C, D), jnp.float32),
            pltpu.VMEM((C, D), jnp.float32),
            pltpu.VMEM_SHARED((NP, D), jnp.float32),
            pltpu.SemaphoreType.DMA,
            pltpu.SemaphoreType.DMA,
        ],
    )
    def k(h_hbm, e0_hbm, e1_hbm, zeros_hbm, out_hbm,
          idx_a, idx_b, buf_a, buf_b, acc, sem_a, sem_b):
        c = lax.axis_index("c")
        s = lax.axis_index("s")
        pltpu.sync_copy(zeros_hbm.at[pl.ds(s * ROWS, ROWS)],
                        acc.at[pl.ds(s * ROWS, ROWS)])
        plsc.subcore_barrier()

        def pipeline(ei_hbm, ch):
            pltpu.sync_copy(ei_hbm.at[s, 0], idx_a)
            pltpu.async_copy(h_hbm.at[idx_a.at[0]], buf_a, sem_a)
            pltpu.sync_copy(ei_hbm.at[s, 1], idx_b)
            pltpu.async_copy(h_hbm.at[idx_b.at[0]], buf_b, sem_b)

            @pl.loop(0, ch - 2, step=2)
            def _(j):
                pltpu.make_async_copy(
                    h_hbm.at[idx_a.at[0]], buf_a, sem_a).wait()
                pltpu.sync_copy(buf_a, acc.at[idx_a.at[1]], add=True)
                pltpu.sync_copy(ei_hbm.at[s, j + 2], idx_a)
                pltpu.async_copy(h_hbm.at[idx_a.at[0]], buf_a, sem_a)
                pltpu.make_async_copy(
                    h_hbm.at[idx_b.at[0]], buf_b, sem_b).wait()
                pltpu.sync_copy(buf_b, acc.at[idx_b.at[1]], add=True)
                pltpu.sync_copy(ei_hbm.at[s, j + 3], idx_b)
                pltpu.async_copy(h_hbm.at[idx_b.at[0]], buf_b, sem_b)

            pltpu.make_async_copy(h_hbm.at[idx_a.at[0]], buf_a, sem_a).wait()
            pltpu.sync_copy(buf_a, acc.at[idx_a.at[1]], add=True)
            pltpu.make_async_copy(h_hbm.at[idx_b.at[0]], buf_b, sem_b).wait()
            pltpu.sync_copy(buf_b, acc.at[idx_b.at[1]], add=True)

        @pl.when(c == 0)
        def _():
            pipeline(e0_hbm, ch0)

        if ch1 > 0:
            @pl.when(c == 1)
            def _():
                pipeline(e1_hbm, ch1)

        plsc.subcore_barrier()
        pltpu.sync_copy(acc.at[pl.ds(s * ROWS, ROWS)],
                        out_hbm.at[c, pl.ds(s * ROWS, ROWS)])

    return k(h, e0_p, e1_p, zeros_mat)


def _tc_matmul(xp, w):
    def body(x_ref, w_ref, o_ref):
        o_ref[...] = jnp.dot(x_ref[...].astype(jnp.bfloat16),
                             w_ref[...].astype(jnp.bfloat16),
                             preferred_element_type=jnp.float32)

    return pl.pallas_call(
        body,
        grid=(NP // BN,),
        in_specs=[pl.BlockSpec((BN, D), lambda i: (i, 0)),
                  pl.BlockSpec((D, D), lambda i: (0, 0))],
        out_specs=pl.BlockSpec((BN, D), lambda i: (i, 0)),
        out_shape=jax.ShapeDtypeStruct((NP, D), jnp.float32),
    )(xp, w)


def _tc_scale(deg_a, deg_b, h1):
    def body(da_ref, db_ref, h_ref, dis_ref, hp_ref):
        dis = lax.rsqrt(1.0 + da_ref[:, 0:1] + db_ref[:, 0:1])
        dis_ref[...] = dis
        hp_ref[...] = h_ref[...] * dis

    return pl.pallas_call(
        body,
        grid=(NP // BN,),
        in_specs=[pl.BlockSpec((BN, D), lambda i: (i, 0)),
                  pl.BlockSpec((BN, D), lambda i: (i, 0)),
                  pl.BlockSpec((BN, D), lambda i: (i, 0))],
        out_specs=[pl.BlockSpec((BN, 1), lambda i: (i, 0)),
                   pl.BlockSpec((BN, D), lambda i: (i, 0))],
        out_shape=[jax.ShapeDtypeStruct((NP, 1), jnp.float32),
                   jax.ShapeDtypeStruct((NP, D), jnp.float32)],
    )(deg_a, deg_b, h1)


def _tc_layer(agg_a, agg_b, hp, dis, b, w):
    def body(aa_ref, ab_ref, hp_ref, dis_ref, b_ref, w_ref, o_ref):
        z = jnp.maximum(
            dis_ref[...] * (aa_ref[...] + ab_ref[...] + hp_ref[...])
            + b_ref[...], 0.0)
        o_ref[...] = jnp.dot(z.astype(jnp.bfloat16),
                             w_ref[...].astype(jnp.bfloat16),
                             preferred_element_type=jnp.float32) * dis_ref[...]

    return pl.pallas_call(
        body,
        grid=(NP // BN,),
        in_specs=[pl.BlockSpec((BN, D), lambda i: (i, 0)),
                  pl.BlockSpec((BN, D), lambda i: (i, 0)),
                  pl.BlockSpec((BN, D), lambda i: (i, 0)),
                  pl.BlockSpec((BN, 1), lambda i: (i, 0)),
                  pl.BlockSpec((1, D), lambda i: (0, 0)),
                  pl.BlockSpec((D, D), lambda i: (0, 0))],
        out_specs=pl.BlockSpec((BN, D), lambda i: (i, 0)),
        out_shape=jax.ShapeDtypeStruct((NP, D), jnp.float32),
    )(agg_a, agg_b, hp, dis, b, w)


def _tc_final(agg_a, agg_b, hp, dis, b):
    def body(aa_ref, ab_ref, hp_ref, dis_ref, b_ref, o_ref):
        o_ref[...] = jnp.maximum(
            dis_ref[...] * (aa_ref[...] + ab_ref[...] + hp_ref[...])
            + b_ref[...], 0.0)

    return pl.pallas_call(
        body,
        grid=(NP // BN,),
        in_specs=[pl.BlockSpec((BN, D), lambda i: (i, 0)),
                  pl.BlockSpec((BN, D), lambda i: (i, 0)),
                  pl.BlockSpec((BN, D), lambda i: (i, 0)),
                  pl.BlockSpec((BN, 1), lambda i: (i, 0)),
                  pl.BlockSpec((1, D), lambda i: (0, 0))],
        out_specs=pl.BlockSpec((BN, D), lambda i: (i, 0)),
        out_shape=jax.ShapeDtypeStruct((NP, D), jnp.float32),
    )(agg_a, agg_b, hp, dis, b)


def kernel(x, edge_index, W1, b1, W2, b2):
    n, _ = x.shape
    e = edge_index.shape[1]
    ch = -(-e // (NW * C))
    ch += ch % 2
    e_pad = NW * ch * C
    ch0 = CH0_FRAC_NUM * 2 * ch // CH0_FRAC_DEN
    ch0 += ch0 % 2
    ch1 = 2 * ch - ch0

    src = edge_index[0]
    dst = edge_index[1]
    pad = e_pad - e
    src_all = jnp.concatenate([src, jnp.zeros((pad,), jnp.int32)])
    dst_all = jnp.concatenate([dst, jnp.full((pad,), n, jnp.int32)])
    dst_p = dst_all.reshape(NW, ch, 1, C)

    t0 = 16 * ch0 * C
    e0_p = jnp.stack([src_all[:t0].reshape(16, ch0, C),
                      dst_all[:t0].reshape(16, ch0, C)], axis=2)
    if ch1 > 0:
        e1_p = jnp.stack([src_all[t0:].reshape(16, ch1, C),
                          dst_all[t0:].reshape(16, ch1, C)], axis=2)
    else:
        e1_p = e0_p[:, :2]

    xp = jnp.pad(x, ((0, NP - n), (0, 0)))
    zeros_mat = jnp.zeros((NP, D), jnp.float32)
    ones_mat = jnp.ones((C, D), jnp.float32)
    b1r = b1.reshape(1, D)
    b2r = b2.reshape(1, D)

    degp = _sc_degree(dst_p, ones_mat, zeros_mat, ch)
    h1 = _tc_matmul(xp, W1)
    dis, h1p = _tc_scale(degp[0], degp[1], h1)
    agg1 = _sc_aggregate(h1p, e0_p, e1_p, zeros_mat, ch0, ch1)
    h2p = _tc_layer(agg1[0], agg1[1], h1p, dis, b1r, W2)
    agg2 = _sc_aggregate(h2p, e0_p, e1_p, zeros_mat, ch0, ch1)
    out = _tc_final(agg2[0], agg2[1], h2p, dis, b2r)
    return out[:n]


# speedup vs baseline: 1.0485x; 1.0485x over previous
import functools

import jax
import jax.numpy as jnp
from jax import lax
from jax.experimental import pallas as pl
from jax.experimental.pallas import tpu as pltpu
from jax.experimental.pallas import tpu_sc as plsc

N = 10000
D = 128
NP = 10240
BN = 2048
NW = 32
C = 128
CH0_FRAC_NUM = 13
CH0_FRAC_DEN = 16
ROWS = NP // 16

_MESH = plsc.VectorSubcoreMesh(core_axis_name="c", subcore_axis_name="s")


def _sc_degree(dst_p, ones_mat, zeros_mat, ch):
    @functools.partial(
        pl.kernel,
        out_type=jax.ShapeDtypeStruct((2, NP, D), jnp.float32),
        mesh=_MESH,
        scratch_types=[
            pltpu.VMEM((1, C), jnp.int32),
            pltpu.VMEM((1, C), jnp.int32),
            pltpu.VMEM((C, D), jnp.float32),
            pltpu.VMEM_SHARED((NP, D), jnp.float32),
            pltpu.SemaphoreType.DMA,
            pltpu.SemaphoreType.DMA,
        ],
    )
    def k(dst_hbm, ones_hbm, zeros_hbm, out_hbm,
          idx_a, idx_b, ones_v, acc, sem_a, sem_b):
        c = lax.axis_index("c")
        s = lax.axis_index("s")
        wid = s * 2 + c
        pltpu.sync_copy(ones_hbm, ones_v)
        pltpu.sync_copy(zeros_hbm.at[pl.ds(s * ROWS, ROWS)],
                        acc.at[pl.ds(s * ROWS, ROWS)])
        plsc.subcore_barrier()

        pltpu.sync_copy(dst_hbm.at[wid, 0], idx_a)
        pltpu.async_copy(ones_v, acc.at[idx_a.at[0]], sem_a, add=True)
        pltpu.sync_copy(dst_hbm.at[wid, 1], idx_b)
        pltpu.async_copy(ones_v, acc.at[idx_b.at[0]], sem_b, add=True)

        @pl.loop(0, ch - 2, step=2)
        def _(j):
            pltpu.make_async_copy(ones_v, acc.at[idx_a.at[0]], sem_a).wait()
            pltpu.sync_copy(dst_hbm.at[wid, j + 2], idx_a)
            pltpu.async_copy(ones_v, acc.at[idx_a.at[0]], sem_a, add=True)
            pltpu.make_async_copy(ones_v, acc.at[idx_b.at[0]], sem_b).wait()
            pltpu.sync_copy(dst_hbm.at[wid, j + 3], idx_b)
            pltpu.async_copy(ones_v, acc.at[idx_b.at[0]], sem_b, add=True)

        pltpu.make_async_copy(ones_v, acc.at[idx_a.at[0]], sem_a).wait()
        pltpu.make_async_copy(ones_v, acc.at[idx_b.at[0]], sem_b).wait()
        plsc.subcore_barrier()
        pltpu.sync_copy(acc.at[pl.ds(s * ROWS, ROWS)],
                        out_hbm.at[c, pl.ds(s * ROWS, ROWS)])

    return k(dst_p, ones_mat, zeros_mat)


def _sc_aggregate(h, e0_p, e1_p, zeros_mat, ch0, ch1):
    @functools.partial(
        pl.kernel,
        out_type=jax.ShapeDtypeStruct((2, NP, D), jnp.float32),
        mesh=_MESH,
        scratch_types=[
            pltpu.VMEM((2, C), jnp.int32),
            pltpu.VMEM((2, C), jnp.int32),
            pltpu.VMEM((C, D), jnp.float32),
            pltpu.VMEM((C, D), jnp.float32),
            pltpu.VMEM_SHARED((NP, D), jnp.float32),
            pltpu.SemaphoreType.DMA,
            pltpu.SemaphoreType.DMA,
        ],
    )
    def k(h_hbm, e0_hbm, e1_hbm, zeros_hbm, out_hbm,
          idx_a, idx_b, buf_a, buf_b, acc, sem_a, sem_b):
        c = lax.axis_index("c")
        s = lax.axis_index("s")
        pltpu.sync_copy(zeros_hbm.at[pl.ds(s * ROWS, ROWS)],
                        acc.at[pl.ds(s * ROWS, ROWS)])
        plsc.subcore_barrier()

        def pipeline(ei_hbm, ch):
            pltpu.sync_copy(ei_hbm.at[s, 0], idx_a)
            pltpu.async_copy(h_hbm.at[idx_a.at[0]], buf_a, sem_a)
            pltpu.sync_copy(ei_hbm.at[s, 1], idx_b)
            pltpu.async_copy(h_hbm.at[idx_b.at[0]], buf_b, sem_b)

            @pl.loop(0, ch - 2, step=2)
            def _(j):
                pltpu.make_async_copy(
                    h_hbm.at[idx_a.at[0]], buf_a, sem_a).wait()
                pltpu.sync_copy(buf_a, acc.at[idx_a.at[1]], add=True)
                pltpu.sync_copy(ei_hbm.at[s, j + 2], idx_a)
                pltpu.async_copy(h_hbm.at[idx_a.at[0]], buf_a, sem_a)
                pltpu.make_async_copy(
                    h_hbm.at[idx_b.at[0]], buf_b, sem_b).wait()
                pltpu.sync_copy(buf_b, acc.at[idx_b.at[1]], add=True)
                pltpu.sync_copy(ei_hbm.at[s, j + 3], idx_b)
                pltpu.async_copy(h_hbm.at[idx_b.at[0]], buf_b, sem_b)

            pltpu.make_async_copy(h_hbm.at[idx_a.at[0]], buf_a, sem_a).wait()
            pltpu.sync_copy(buf_a, acc.at[idx_a.at[1]], add=True)
            pltpu.make_async_copy(h_hbm.at[idx_b.at[0]], buf_b, sem_b).wait()
            pltpu.sync_copy(buf_b, acc.at[idx_b.at[1]], add=True)

        @pl.when(c == 0)
        def _():
            pipeline(e0_hbm, ch0)

        if ch1 > 0:
            @pl.when(c == 1)
            def _():
                pipeline(e1_hbm, ch1)

        plsc.subcore_barrier()
        pltpu.sync_copy(acc.at[pl.ds(s * ROWS, ROWS)],
                        out_hbm.at[c, pl.ds(s * ROWS, ROWS)])

    return k(h, e0_p, e1_p, zeros_mat)


def _tc_matmul(xp, w):
    def body(x_ref, w_ref, o_ref):
        o_ref[...] = jnp.dot(x_ref[...].astype(jnp.bfloat16),
                             w_ref[...].astype(jnp.bfloat16),
                             preferred_element_type=jnp.float32)

    return pl.pallas_call(
        body,
        grid=(NP // BN,),
        in_specs=[pl.BlockSpec((BN, D), lambda i: (i, 0)),
                  pl.BlockSpec((D, D), lambda i: (0, 0))],
        out_specs=pl.BlockSpec((BN, D), lambda i: (i, 0)),
        out_shape=jax.ShapeDtypeStruct((NP, D), jnp.float32),
    )(xp, w)


def _tc_scale(deg_a, deg_b, h1):
    def body(da_ref, db_ref, h_ref, dis_ref, hp_ref):
        dis = lax.rsqrt(1.0 + da_ref[:, 0:1] + db_ref[:, 0:1])
        dis_ref[...] = dis
        hp_ref[...] = h_ref[...] * dis

    return pl.pallas_call(
        body,
        grid=(NP // BN,),
        in_specs=[pl.BlockSpec((BN, D), lambda i: (i, 0)),
                  pl.BlockSpec((BN, D), lambda i: (i, 0)),
                  pl.BlockSpec((BN, D), lambda i: (i, 0))],
        out_specs=[pl.BlockSpec((BN, 1), lambda i: (i, 0)),
                   pl.BlockSpec((BN, D), lambda i: (i, 0))],
        out_shape=[jax.ShapeDtypeStruct((NP, 1), jnp.float32),
                   jax.ShapeDtypeStruct((NP, D), jnp.float32)],
    )(deg_a, deg_b, h1)


def _tc_layer(agg_a, agg_b, hp, dis, b, w):
    def body(aa_ref, ab_ref, hp_ref, dis_ref, b_ref, w_ref, o_ref):
        z = jnp.maximum(
            dis_ref[...] * (aa_ref[...] + ab_ref[...] + hp_ref[...])
            + b_ref[...], 0.0)
        o_ref[...] = jnp.dot(z.astype(jnp.bfloat16),
                             w_ref[...].astype(jnp.bfloat16),
                             preferred_element_type=jnp.float32) * dis_ref[...]

    return pl.pallas_call(
        body,
        grid=(NP // BN,),
        in_specs=[pl.BlockSpec((BN, D), lambda i: (i, 0)),
                  pl.BlockSpec((BN, D), lambda i: (i, 0)),
                  pl.BlockSpec((BN, D), lambda i: (i, 0)),
                  pl.BlockSpec((BN, 1), lambda i: (i, 0)),
                  pl.BlockSpec((1, D), lambda i: (0, 0)),
                  pl.BlockSpec((D, D), lambda i: (0, 0))],
        out_specs=pl.BlockSpec((BN, D), lambda i: (i, 0)),
        out_shape=jax.ShapeDtypeStruct((NP, D), jnp.float32),
    )(agg_a, agg_b, hp, dis, b, w)


def _tc_final(agg_a, agg_b, hp, dis, b):
    def body(aa_ref, ab_ref, hp_ref, dis_ref, b_ref, o_ref):
        o_ref[...] = jnp.maximum(
            dis_ref[...] * (aa_ref[...] + ab_ref[...] + hp_ref[...])
            + b_ref[...], 0.0)

    return pl.pallas_call(
        body,
        grid=(NP // BN,),
        in_specs=[pl.BlockSpec((BN, D), lambda i: (i, 0)),
                  pl.BlockSpec((BN, D), lambda i: (i, 0)),
                  pl.BlockSpec((BN, D), lambda i: (i, 0)),
                  pl.BlockSpec((BN, 1), lambda i: (i, 0)),
                  pl.BlockSpec((1, D), lambda i: (0, 0))],
        out_specs=pl.BlockSpec((BN, D), lambda i: (i, 0)),
        out_shape=jax.ShapeDtypeStruct((NP, D), jnp.float32),
    )(agg_a, agg_b, hp, dis, b)


def kernel(x, edge_index, W1, b1, W2, b2):
    n, _ = x.shape
    e = edge_index.shape[1]
    ch = -(-e // (NW * C))
    ch += ch % 2
    e_pad = NW * ch * C
    ch0 = CH0_FRAC_NUM * 2 * ch // CH0_FRAC_DEN
    ch0 += ch0 % 2
    ch1 = 2 * ch - ch0

    src = edge_index[0]
    dst = edge_index[1]
    pad = e_pad - e
    src_all = jnp.concatenate([src, jnp.zeros((pad,), jnp.int32)])
    dst_all = jnp.concatenate([dst, jnp.full((pad,), n, jnp.int32)])
    dst_p = dst_all.reshape(NW, ch, 1, C)

    t0 = 16 * ch0 * C
    e0_p = jnp.stack([src_all[:t0].reshape(16, ch0, C),
                      dst_all[:t0].reshape(16, ch0, C)], axis=2)
    if ch1 > 0:
        e1_p = jnp.stack([src_all[t0:].reshape(16, ch1, C),
                          dst_all[t0:].reshape(16, ch1, C)], axis=2)
    else:
        e1_p = e0_p[:, :2]

    xp = jnp.pad(x, ((0, NP - n), (0, 0)))
    zeros_mat = jnp.zeros((NP, D), jnp.float32)
    ones_mat = jnp.ones((C, D), jnp.float32)
    b1r = b1.reshape(1, D)
    b2r = b2.reshape(1, D)

    degp = _sc_degree(dst_p, ones_mat, zeros_mat, ch)
    h1 = _tc_matmul(xp, W1)
    dis, h1p = _tc_scale(degp[0], degp[1], h1)
    agg1 = _sc_aggregate(h1p, e0_p, e1_p, zeros_mat, ch0, ch1)
    h2p = _tc_layer(agg1[0], agg1[1], h1p, dis, b1r, W2)
    agg2 = _sc_aggregate(h2p, e0_p, e1_p, zeros_mat, ch0, ch1)
    out = _tc_final(agg2[0], agg2[1], h2p, dis, b2r)
    return out[:n]
